# Initial kernel scaffold; baseline (speedup 1.0000x reference)
import jax
import jax.numpy as jnp
from jax.experimental import pallas as pl


def kernel(table, seq_len, layer_attention_span):
    raise NotImplementedError("write your pallas kernel here")


# speedup vs baseline: 7.5661x; 7.5661x over previous
import jax
import jax.numpy as jnp
from jax import lax
from jax.experimental import pallas as pl
from jax.experimental.pallas import tpu as pltpu
from jax.experimental.pallas import tpu_sc as plsc

SEQ = 2048
EMB = 32
TMPL = 2 * SEQ
NC, NS = 2, 16
NW = NC * NS
ROWS_PER_W = SEQ // NW
TROWS_PER_S = TMPL // NS
GCHUNK = 128


def _sc_body(idx_h, table_h, out_h, idx_v, rows_v, tmpl_sh, sem):
    c = lax.axis_index("c")
    s = lax.axis_index("s")

    for chunk in range(TROWS_PER_S // GCHUNK):
        base = s * TROWS_PER_S + chunk * GCHUNK
        pltpu.sync_copy(idx_h.at[pl.ds(base, GCHUNK)], idx_v)
        pltpu.async_copy(table_h.at[idx_v], rows_v, sem).wait()
        pltpu.sync_copy(rows_v, tmpl_sh.at[pl.ds(base, GCHUNK)])
    plsc.subcore_barrier()

    wid = s * NC + c

    def write_row(r, carry):
        i = wid * ROWS_PER_W + r
        start = (SEQ - 1) - i
        pltpu.sync_copy(tmpl_sh.at[pl.ds(start, SEQ)], out_h.at[i])
        return carry

    lax.fori_loop(0, ROWS_PER_W, write_row, 0)


_sc_call = pl.kernel(
    _sc_body,
    out_type=jax.ShapeDtypeStruct((SEQ, SEQ, EMB), jnp.float32),
    mesh=plsc.VectorSubcoreMesh(core_axis_name="c", subcore_axis_name="s"),
    scratch_types=[
        pltpu.VMEM((GCHUNK,), jnp.int32),
        pltpu.VMEM((GCHUNK, EMB), jnp.float32),
        pltpu.VMEM_SHARED((TMPL, EMB), jnp.float32),
        pltpu.SemaphoreType.DMA,
    ],
    compiler_params=pltpu.CompilerParams(use_tc_tiling_on_sc=False),
)


def kernel(table, seq_len, layer_attention_span):
    span = jnp.asarray(layer_attention_span, jnp.int32)
    k = jnp.arange(TMPL, dtype=jnp.int32)
    idx = jnp.clip(jnp.abs(k - (SEQ - 1)), 0, span)
    return _sc_call(idx, table)
